# baseline (speedup 1.0000x reference)
import functools

import jax
import jax.numpy as jnp
from jax import lax
from jax.experimental import pallas as pl
from jax.experimental.pallas import tpu as pltpu
from jax.experimental.pallas import tpu_sc as plsc

N = 10000
E = 320000
F = 128
N2 = 10240
NC, NS = 2, 16
NW = NC * NS
CHUNK = 128
EPT = 10240
KCH = EPT // CHUNK
NBUF = 4
NGRP = KCH // NBUF
RPT = N2 // NS

_mesh = plsc.VectorSubcoreMesh(core_axis_name="c", subcore_axis_name="s")


@functools.partial(
    pl.kernel,
    out_type=jax.ShapeDtypeStruct((NC, N2), jnp.float32),
    mesh=_mesh,
    scratch_types=[
        pltpu.VMEM((KCH, CHUNK), jnp.int32),
        pltpu.VMEM((KCH, CHUNK), jnp.float32),
        pltpu.VMEM_SHARED((N2,), jnp.float32),
    ],
)
def _degree(col_hbm, ew_hbm, zeros1_hbm, out_hbm, colv, eww, dacc):
    c = lax.axis_index("c")
    s = lax.axis_index("s")
    w = c * NS + s
    base = s * RPT
    pltpu.sync_copy(zeros1_hbm.at[pl.ds(base, RPT)], dacc.at[pl.ds(base, RPT)])
    pltpu.sync_copy(col_hbm.at[w], colv)
    pltpu.sync_copy(ew_hbm.at[w], eww)
    plsc.subcore_barrier()

    def body(j, carry):
        pltpu.sync_copy(eww.at[j], dacc.at[colv.at[j]], add=True)
        return carry

    lax.fori_loop(0, KCH, body, 0)
    plsc.subcore_barrier()
    pltpu.sync_copy(dacc.at[pl.ds(base, RPT)], out_hbm.at[c, pl.ds(base, RPT)])


@functools.partial(
    pl.kernel,
    out_type=jax.ShapeDtypeStruct((NC, N2, F), jnp.float32),
    mesh=_mesh,
    scratch_types=[
        pltpu.VMEM((KCH, CHUNK), jnp.int32),
        pltpu.VMEM((KCH, CHUNK), jnp.int32),
        pltpu.VMEM((KCH, CHUNK), jnp.float32),
    ]
    + [pltpu.VMEM((CHUNK, F), jnp.float32)]
    + [pltpu.VMEM_SHARED((N2, F), jnp.float32)],
)
def _msg_pass(h_hbm, row_hbm, col_hbm, ew_hbm, zeros2_hbm, out_hbm,
              rowv, colv, eww, rows0, acc):
    rows = (rows0,)
    c = lax.axis_index("c")
    s = lax.axis_index("s")
    w = c * NS + s
    base = s * RPT
    pltpu.sync_copy(zeros2_hbm.at[pl.ds(base, RPT)], acc.at[pl.ds(base, RPT)])
    pltpu.sync_copy(row_hbm.at[w], rowv)
    pltpu.sync_copy(col_hbm.at[w], colv)
    pltpu.sync_copy(ew_hbm.at[w], eww)
    plsc.subcore_barrier()

    def scale(buf, j):
        def scale_body(blk, c2):
            ew16 = eww[j, pl.ds(blk * 16, 16)]
            for l in range(16):
                wgt = ew16[l]
                i = blk * 16 + l
                for f in range(F // 16):
                    sl = pl.ds(f * 16, 16)
                    buf[i, sl] = buf[i, sl] * wgt
            return c2

        lax.fori_loop(0, CHUNK // 16, scale_body, 0)

    def _chunk(j, carry):
        pltpu.sync_copy(h_hbm.at[rowv.at[j]], rows[0])
        scale(rows[0], j)
        pltpu.sync_copy(rows[0], acc.at[colv.at[j]], add=True)
        return carry

    lax.fori_loop(0, KCH, _chunk, 0)

    plsc.subcore_barrier()
    pltpu.sync_copy(acc.at[pl.ds(base, RPT)], out_hbm.at[c, pl.ds(base, RPT)])


RB = 2048
GRID = N2 // RB


def _tc1_body(deg_ref, x_ref, w_ref, h1p_ref, dinv_ref):
    dsum = 1.0 + deg_ref[:, 0:1] + deg_ref[:, 1:2]
    dinv = lax.rsqrt(dsum)
    h = jnp.dot(x_ref[...], w_ref[...], preferred_element_type=jnp.float32)
    h1p_ref[...] = h * dinv
    dinv_ref[...] = dinv


def _tc_mid_body(p_ref, h1p_ref, dinv_ref, b1_ref, w23_ref, h23p_ref):
    p = p_ref[...]
    dinv = dinv_ref[...]
    out1 = dinv * (p[0] + p[1] + h1p_ref[...]) + b1_ref[...]
    h = jnp.maximum(out1, 0.0)
    h23p_ref[...] = jnp.dot(h, w23_ref[...],
                            preferred_element_type=jnp.float32) * dinv


def _tc_fin_body(q_ref, h23p_ref, dinv_ref, b23_ref, out_ref):
    q = q_ref[...]
    out_ref[...] = dinv_ref[...] * (q[0] + q[1] + h23p_ref[...]) + b23_ref[...]


def _tc1(degT, x2, W1):
    return pl.pallas_call(
        _tc1_body,
        grid=(GRID,),
        in_specs=[
            pl.BlockSpec((RB, 2), lambda i: (i, 0)),
            pl.BlockSpec((RB, F), lambda i: (i, 0)),
            pl.BlockSpec((F, F), lambda i: (0, 0)),
        ],
        out_specs=[
            pl.BlockSpec((RB, F), lambda i: (i, 0)),
            pl.BlockSpec((RB, 1), lambda i: (i, 0)),
        ],
        out_shape=[
            jax.ShapeDtypeStruct((N2, F), jnp.float32),
            jax.ShapeDtypeStruct((N2, 1), jnp.float32),
        ],
    )(degT, x2, W1)


def _tc_mid(P, h1p, dinv, b1r, W23):
    return pl.pallas_call(
        _tc_mid_body,
        grid=(GRID,),
        in_specs=[
            pl.BlockSpec((NC, RB, F), lambda i: (0, i, 0)),
            pl.BlockSpec((RB, F), lambda i: (i, 0)),
            pl.BlockSpec((RB, 1), lambda i: (i, 0)),
            pl.BlockSpec((1, F), lambda i: (0, 0)),
            pl.BlockSpec((F, F), lambda i: (0, 0)),
        ],
        out_specs=pl.BlockSpec((RB, F), lambda i: (i, 0)),
        out_shape=jax.ShapeDtypeStruct((N2, F), jnp.float32),
    )(P, h1p, dinv, b1r, W23)


def _tc_fin(Q, h23p, dinv, b23r):
    return pl.pallas_call(
        _tc_fin_body,
        grid=(GRID,),
        in_specs=[
            pl.BlockSpec((NC, RB, F), lambda i: (0, i, 0)),
            pl.BlockSpec((RB, F), lambda i: (i, 0)),
            pl.BlockSpec((RB, 1), lambda i: (i, 0)),
            pl.BlockSpec((1, F), lambda i: (0, 0)),
        ],
        out_specs=pl.BlockSpec((RB, F), lambda i: (i, 0)),
        out_shape=jax.ShapeDtypeStruct((N2, F), jnp.float32),
    )(Q, h23p, dinv, b23r)


def kernel(x, edges, weights, W1, b1, W2, b2, W3, b3):
    row = edges[0]
    col = edges[1]
    pad = NW * EPT - E
    rowp = jnp.concatenate([row, jnp.zeros((pad,), row.dtype)]).reshape(NW, KCH, CHUNK)
    colp = jnp.concatenate([col, jnp.zeros((pad,), col.dtype)]).reshape(NW, KCH, CHUNK)
    ewp = jnp.concatenate([weights, jnp.zeros((pad,), weights.dtype)]).reshape(NW, KCH, CHUNK)
    x2 = jnp.pad(x, ((0, N2 - N), (0, 0)))
    zeros2 = jnp.zeros((N2, F), jnp.float32)
    zeros1 = jnp.zeros((N2,), jnp.float32)
    W23 = jnp.concatenate([W2, W3], axis=1)
    b23r = jnp.concatenate([b2, b3]).reshape(1, F)
    b1r = b1.reshape(1, F)

    degp = _degree(colp, ewp, zeros1)
    degT = degp.T
    h1p, dinv = _tc1(degT, x2, W1)
    P = _msg_pass(h1p, rowp, colp, ewp, zeros2)
    h23p = _tc_mid(P, h1p, dinv, b1r, W23)
    Q = _msg_pass(h23p, rowp, colp, ewp, zeros2)
    out = _tc_fin(Q, h23p, dinv, b23r)
    return out[:N, :64], out[:N, 64:]


# speedup vs baseline: 1.4146x; 1.4146x over previous
import functools

import jax
import jax.numpy as jnp
from jax import lax
from jax.experimental import pallas as pl
from jax.experimental.pallas import tpu as pltpu
from jax.experimental.pallas import tpu_sc as plsc

N = 10000
E = 320000
F = 128
N2 = 10240
NC, NS = 2, 16
NW = NC * NS
CHUNK = 128
EPT = 10112
KCH = EPT // CHUNK
NBUF = 4
NGRP = KCH // NBUF
RPT = N2 // NS

_mesh = plsc.VectorSubcoreMesh(core_axis_name="c", subcore_axis_name="s")


@functools.partial(
    pl.kernel,
    out_type=jax.ShapeDtypeStruct((NC, N2), jnp.float32),
    mesh=_mesh,
    scratch_types=[
        pltpu.VMEM((KCH, CHUNK), jnp.int32),
        pltpu.VMEM((KCH, CHUNK), jnp.float32),
        pltpu.VMEM_SHARED((N2,), jnp.float32),
    ],
)
def _degree(col_hbm, ew_hbm, zeros1_hbm, out_hbm, colv, eww, dacc):
    c = lax.axis_index("c")
    s = lax.axis_index("s")
    w = c * NS + s
    base = s * RPT
    pltpu.sync_copy(zeros1_hbm.at[pl.ds(base, RPT)], dacc.at[pl.ds(base, RPT)])
    pltpu.sync_copy(col_hbm.at[w], colv)
    pltpu.sync_copy(ew_hbm.at[w], eww)
    plsc.subcore_barrier()

    def body(j, carry):
        pltpu.sync_copy(eww.at[j], dacc.at[colv.at[j]], add=True)
        return carry

    lax.fori_loop(0, KCH, body, 0)
    plsc.subcore_barrier()
    pltpu.sync_copy(dacc.at[pl.ds(base, RPT)], out_hbm.at[c, pl.ds(base, RPT)])


@functools.partial(
    pl.kernel,
    out_type=jax.ShapeDtypeStruct((NC, N2, F), jnp.float32),
    mesh=_mesh,
    scratch_types=[
        pltpu.VMEM((KCH, CHUNK), jnp.int32),
        pltpu.VMEM((KCH, CHUNK), jnp.int32),
        pltpu.VMEM((KCH, CHUNK), jnp.float32),
    ]
    + [pltpu.VMEM((CHUNK, F), jnp.float32)]
    + [pltpu.VMEM_SHARED((N2, F), jnp.float32)],
)
def _msg_pass(h_hbm, row_hbm, col_hbm, ew_hbm, zeros2_hbm, out_hbm,
              rowv, colv, eww, rows0, acc):
    rows = (rows0,)
    c = lax.axis_index("c")
    s = lax.axis_index("s")
    w = c * NS + s
    base = s * RPT
    pltpu.sync_copy(zeros2_hbm.at[pl.ds(base, RPT)], acc.at[pl.ds(base, RPT)])
    pltpu.sync_copy(row_hbm.at[w], rowv)
    pltpu.sync_copy(col_hbm.at[w], colv)
    pltpu.sync_copy(ew_hbm.at[w], eww)
    plsc.subcore_barrier()

    def scale(buf, j):
        def scale_body(blk, c2):
            ew16 = eww[j, pl.ds(blk * 16, 16)]
            for l in range(16):
                wgt = ew16[l]
                i = blk * 16 + l
                for f in range(F // 16):
                    sl = pl.ds(f * 16, 16)
                    buf[i, sl] = buf[i, sl] * wgt
            return c2

        lax.fori_loop(0, CHUNK // 16, scale_body, 0)

    def _chunk(j, carry):
        pltpu.sync_copy(h_hbm.at[rowv.at[j]], rows[0])
        scale(rows[0], j)
        pltpu.sync_copy(rows[0], acc.at[colv.at[j]], add=True)
        return carry

    lax.fori_loop(0, KCH, _chunk, 0)

    plsc.subcore_barrier()
    pltpu.sync_copy(acc.at[pl.ds(base, RPT)], out_hbm.at[c, pl.ds(base, RPT)])


RB = 2048
GRID = N2 // RB


def _tc1_body(deg_ref, x_ref, w_ref, h1p_ref, dinv_ref):
    dsum = 1.0 + deg_ref[:, 0:1] + deg_ref[:, 1:2]
    dinv = lax.rsqrt(dsum)
    h = jnp.dot(x_ref[...], w_ref[...], preferred_element_type=jnp.float32)
    h1p_ref[...] = h * dinv
    dinv_ref[...] = dinv


def _tc_mid_body(p_ref, h1p_ref, dinv_ref, b1_ref, w23_ref, h23p_ref):
    p = p_ref[...]
    dinv = dinv_ref[...]
    out1 = dinv * (p[0] + p[1] + h1p_ref[...]) + b1_ref[...]
    h = jnp.maximum(out1, 0.0)
    h23p_ref[...] = jnp.dot(h, w23_ref[...],
                            preferred_element_type=jnp.float32) * dinv


def _tc_fin_body(q_ref, h23p_ref, dinv_ref, b23_ref, out_ref):
    q = q_ref[...]
    out_ref[...] = dinv_ref[...] * (q[0] + q[1] + h23p_ref[...]) + b23_ref[...]


def _tc1(degT, x2, W1):
    return pl.pallas_call(
        _tc1_body,
        grid=(GRID,),
        in_specs=[
            pl.BlockSpec((RB, 2), lambda i: (i, 0)),
            pl.BlockSpec((RB, F), lambda i: (i, 0)),
            pl.BlockSpec((F, F), lambda i: (0, 0)),
        ],
        out_specs=[
            pl.BlockSpec((RB, F), lambda i: (i, 0)),
            pl.BlockSpec((RB, 1), lambda i: (i, 0)),
        ],
        out_shape=[
            jax.ShapeDtypeStruct((N2, F), jnp.float32),
            jax.ShapeDtypeStruct((N2, 1), jnp.float32),
        ],
    )(degT, x2, W1)


def _tc_mid(P, h1p, dinv, b1r, W23):
    return pl.pallas_call(
        _tc_mid_body,
        grid=(GRID,),
        in_specs=[
            pl.BlockSpec((NC, RB, F), lambda i: (0, i, 0)),
            pl.BlockSpec((RB, F), lambda i: (i, 0)),
            pl.BlockSpec((RB, 1), lambda i: (i, 0)),
            pl.BlockSpec((1, F), lambda i: (0, 0)),
            pl.BlockSpec((F, F), lambda i: (0, 0)),
        ],
        out_specs=pl.BlockSpec((RB, F), lambda i: (i, 0)),
        out_shape=jax.ShapeDtypeStruct((N2, F), jnp.float32),
    )(P, h1p, dinv, b1r, W23)


def _tc_fin(Q, h23p, dinv, b23r):
    return pl.pallas_call(
        _tc_fin_body,
        grid=(GRID,),
        in_specs=[
            pl.BlockSpec((NC, RB, F), lambda i: (0, i, 0)),
            pl.BlockSpec((RB, F), lambda i: (i, 0)),
            pl.BlockSpec((RB, 1), lambda i: (i, 0)),
            pl.BlockSpec((1, F), lambda i: (0, 0)),
        ],
        out_specs=pl.BlockSpec((RB, F), lambda i: (i, 0)),
        out_shape=jax.ShapeDtypeStruct((N2, F), jnp.float32),
    )(Q, h23p, dinv, b23r)


def kernel(x, edges, weights, W1, b1, W2, b2, W3, b3):
    row = edges[0]
    col = edges[1]
    pad = NW * EPT - E
    rowp = jnp.concatenate([row, jnp.zeros((pad,), row.dtype)]).reshape(NW, KCH, CHUNK)
    colp = jnp.concatenate([col, jnp.zeros((pad,), col.dtype)]).reshape(NW, KCH, CHUNK)
    ewp = jnp.concatenate([weights, jnp.zeros((pad,), weights.dtype)]).reshape(NW, KCH, CHUNK)
    x2 = jnp.pad(x, ((0, N2 - N), (0, 0)))
    zeros2 = jnp.zeros((N2, F), jnp.float32)
    zeros1 = jnp.zeros((N2,), jnp.float32)
    W23 = jnp.concatenate([W2, W3], axis=1)
    b23r = jnp.concatenate([b2, b3]).reshape(1, F)
    b1r = b1.reshape(1, F)

    degp = _degree(colp, ewp, zeros1)
    degT = degp.T
    h1p, dinv = _tc1(degT, x2, W1)
    P = _msg_pass(h1p, rowp, colp, ewp, zeros2)
    h23p = _tc_mid(P, h1p, dinv, b1r, W23)
    Q = _msg_pass(h23p, rowp, colp, ewp, zeros2)
    out = _tc_fin(Q, h23p, dinv, b23r)
    return out[:N, :64], out[:N, 64:]


# speedup vs baseline: 2.1305x; 1.5061x over previous
import functools

import jax
import jax.numpy as jnp
from jax import lax
from jax.experimental import pallas as pl
from jax.experimental.pallas import tpu as pltpu
from jax.experimental.pallas import tpu_sc as plsc

N = 10000
E = 320000
F = 128
N2 = 10240
NC, NS = 2, 16
NW = NC * NS
CHUNK = 128
EPT = 10240
KCH = EPT // CHUNK
NBUF = 4
NGRP = KCH // NBUF
RPT = N2 // NS

_mesh = plsc.VectorSubcoreMesh(core_axis_name="c", subcore_axis_name="s")


@functools.partial(
    pl.kernel,
    out_type=jax.ShapeDtypeStruct((NC, N2), jnp.float32),
    mesh=_mesh,
    scratch_types=[
        pltpu.VMEM((KCH, CHUNK), jnp.int32),
        pltpu.VMEM((KCH, CHUNK), jnp.float32),
        pltpu.VMEM_SHARED((N2,), jnp.float32),
    ],
)
def _degree(col_hbm, ew_hbm, zeros1_hbm, out_hbm, colv, eww, dacc):
    c = lax.axis_index("c")
    s = lax.axis_index("s")
    w = c * NS + s
    base = s * RPT
    pltpu.sync_copy(zeros1_hbm.at[pl.ds(base, RPT)], dacc.at[pl.ds(base, RPT)])
    pltpu.sync_copy(col_hbm.at[w], colv)
    pltpu.sync_copy(ew_hbm.at[w], eww)
    plsc.subcore_barrier()

    def body(j, carry):
        pltpu.sync_copy(eww.at[j], dacc.at[colv.at[j]], add=True)
        return carry

    lax.fori_loop(0, KCH, body, 0)
    plsc.subcore_barrier()
    pltpu.sync_copy(dacc.at[pl.ds(base, RPT)], out_hbm.at[c, pl.ds(base, RPT)])


@functools.partial(
    pl.kernel,
    out_type=jax.ShapeDtypeStruct((NC, N2, F), jnp.float32),
    mesh=_mesh,
    scratch_types=[
        pltpu.VMEM((KCH, CHUNK), jnp.int32),
        pltpu.VMEM((KCH, CHUNK), jnp.int32),
        pltpu.VMEM((KCH, CHUNK), jnp.float32),
    ]
    + [pltpu.VMEM((CHUNK, F), jnp.float32)]
    + [pltpu.VMEM_SHARED((N2, F), jnp.float32)],
)
def _msg_pass(h_hbm, row_hbm, col_hbm, ew_hbm, zeros2_hbm, out_hbm,
              rowv, colv, eww, rows0, acc):
    rows = (rows0,)
    c = lax.axis_index("c")
    s = lax.axis_index("s")
    w = c * NS + s
    base = s * RPT
    pltpu.sync_copy(zeros2_hbm.at[pl.ds(base, RPT)], acc.at[pl.ds(base, RPT)])
    pltpu.sync_copy(row_hbm.at[w], rowv)
    pltpu.sync_copy(col_hbm.at[w], colv)
    pltpu.sync_copy(ew_hbm.at[w], eww)
    plsc.subcore_barrier()

    def scale(buf, j):
        def scale_body(blk, c2):
            ew16 = eww[j, pl.ds(blk * 16, 16)]
            for l in range(16):
                wgt = ew16[l]
                i = blk * 16 + l
                for f in range(F // 16):
                    sl = pl.ds(f * 16, 16)
                    buf[i, sl] = buf[i, sl] * wgt
            return c2

        lax.fori_loop(0, CHUNK // 16, scale_body, 0)

    def _chunk(j, carry):
        pltpu.sync_copy(h_hbm.at[rowv.at[j]], rows[0])
        scale(rows[0], j)
        pltpu.sync_copy(rows[0], acc.at[colv.at[j]], add=True)
        return carry

    lax.fori_loop(0, KCH, _chunk, 0)

    plsc.subcore_barrier()
    pltpu.sync_copy(acc.at[pl.ds(base, RPT)], out_hbm.at[c, pl.ds(base, RPT)])


RB = 2048
GRID = N2 // RB


def _tc1_body(deg_ref, x_ref, w_ref, h1p_ref, dinv_ref):
    dsum = 1.0 + deg_ref[:, 0:1] + deg_ref[:, 1:2]
    dinv = lax.rsqrt(dsum)
    h = jnp.dot(x_ref[...], w_ref[...], preferred_element_type=jnp.float32)
    h1p_ref[...] = h * dinv
    dinv_ref[...] = dinv


def _tc_mid_body(p_ref, h1p_ref, dinv_ref, b1_ref, w23_ref, h23p_ref):
    p = p_ref[...]
    dinv = dinv_ref[...]
    out1 = dinv * (p[0] + p[1] + h1p_ref[...]) + b1_ref[...]
    h = jnp.maximum(out1, 0.0)
    h23p_ref[...] = jnp.dot(h, w23_ref[...],
                            preferred_element_type=jnp.float32) * dinv


def _tc_fin_body(q_ref, h23p_ref, dinv_ref, b23_ref, out_ref):
    q = q_ref[...]
    out_ref[...] = dinv_ref[...] * (q[0] + q[1] + h23p_ref[...]) + b23_ref[...]


def _tc1(degT, x2, W1):
    return pl.pallas_call(
        _tc1_body,
        grid=(GRID,),
        in_specs=[
            pl.BlockSpec((RB, 2), lambda i: (i, 0)),
            pl.BlockSpec((RB, F), lambda i: (i, 0)),
            pl.BlockSpec((F, F), lambda i: (0, 0)),
        ],
        out_specs=[
            pl.BlockSpec((RB, F), lambda i: (i, 0)),
            pl.BlockSpec((RB, 1), lambda i: (i, 0)),
        ],
        out_shape=[
            jax.ShapeDtypeStruct((N2, F), jnp.float32),
            jax.ShapeDtypeStruct((N2, 1), jnp.float32),
        ],
    )(degT, x2, W1)


def _tc_mid(P, h1p, dinv, b1r, W23):
    return pl.pallas_call(
        _tc_mid_body,
        grid=(GRID,),
        in_specs=[
            pl.BlockSpec((NC, RB, F), lambda i: (0, i, 0)),
            pl.BlockSpec((RB, F), lambda i: (i, 0)),
            pl.BlockSpec((RB, 1), lambda i: (i, 0)),
            pl.BlockSpec((1, F), lambda i: (0, 0)),
            pl.BlockSpec((F, F), lambda i: (0, 0)),
        ],
        out_specs=pl.BlockSpec((RB, F), lambda i: (i, 0)),
        out_shape=jax.ShapeDtypeStruct((N2, F), jnp.float32),
    )(P, h1p, dinv, b1r, W23)


def _tc_fin(Q, h23p, dinv, b23r):
    return pl.pallas_call(
        _tc_fin_body,
        grid=(GRID,),
        in_specs=[
            pl.BlockSpec((NC, RB, F), lambda i: (0, i, 0)),
            pl.BlockSpec((RB, F), lambda i: (i, 0)),
            pl.BlockSpec((RB, 1), lambda i: (i, 0)),
            pl.BlockSpec((1, F), lambda i: (0, 0)),
        ],
        out_specs=pl.BlockSpec((RB, F), lambda i: (i, 0)),
        out_shape=jax.ShapeDtypeStruct((N2, F), jnp.float32),
    )(Q, h23p, dinv, b23r)


def kernel(x, edges, weights, W1, b1, W2, b2, W3, b3):
    row = edges[0]
    col = edges[1]
    pad = NW * EPT - E
    spread = jnp.arange(pad, dtype=row.dtype) % N
    rowp = jnp.concatenate([row, spread]).reshape(NW, KCH, CHUNK)
    colp = jnp.concatenate([col, spread]).reshape(NW, KCH, CHUNK)
    ewp = jnp.concatenate([weights, jnp.zeros((pad,), weights.dtype)]).reshape(NW, KCH, CHUNK)
    x2 = jnp.pad(x, ((0, N2 - N), (0, 0)))
    zeros2 = jnp.zeros((N2, F), jnp.float32)
    zeros1 = jnp.zeros((N2,), jnp.float32)
    W23 = jnp.concatenate([W2, W3], axis=1)
    b23r = jnp.concatenate([b2, b3]).reshape(1, F)
    b1r = b1.reshape(1, F)

    degp = _degree(colp, ewp, zeros1)
    degT = degp.T
    h1p, dinv = _tc1(degT, x2, W1)
    P = _msg_pass(h1p, rowp, colp, ewp, zeros2)
    h23p = _tc_mid(P, h1p, dinv, b1r, W23)
    Q = _msg_pass(h23p, rowp, colp, ewp, zeros2)
    out = _tc_fin(Q, h23p, dinv, b23r)
    return out[:N, :64], out[:N, 64:]
